# baseline (speedup 1.0000x reference)
import functools
import jax
import jax.numpy as jnp
from jax import lax
from jax.experimental import pallas as pl
from jax.experimental.pallas import tpu as pltpu
from jax.experimental.pallas import tpu_sc as plsc

N = 10000
NP = 10240
NPT = NP // 16
NC = 2
NS = 16
NW = NC * NS
CHUNK = 128

@functools.lru_cache(maxsize=None)
def _mesh():
    return plsc.VectorSubcoreMesh(core_axis_name="c", subcore_axis_name="s")


def _pad_edges(ei, n_chunks_total):
    e = ei.shape[1]
    pad = n_chunks_total * CHUNK - e
    row = jnp.concatenate([ei[0], jnp.zeros((pad,), jnp.int32)])
    col = jnp.concatenate([ei[1], jnp.full((pad,), N, jnp.int32)])
    return (row.reshape(n_chunks_total, CHUNK),
            col.reshape(n_chunks_total, CHUNK))


@functools.lru_cache(maxsize=None)
def _make_deg_kernel(c_per_tile):
    @functools.partial(
        pl.kernel,
        mesh=_mesh(),
        out_type=jax.ShapeDtypeStruct((NC * 3 * NP,), jnp.float32),
        scratch_types=[
            pltpu.VMEM((c_per_tile, CHUNK), jnp.int32),
            pltpu.VMEM((CHUNK,), jnp.float32),
            pltpu.VMEM_SHARED((3 * NP,), jnp.float32),
        ],
    )
    def k(cols_hbm, ones_hbm, zeros_hbm, out_hbm, idx_v, ones_v, acc_sh):
        cid = lax.axis_index("c")
        sid = lax.axis_index("s")
        wid = sid * NC + cid
        pltpu.sync_copy(zeros_hbm, acc_sh.at[pl.ds(sid * 3 * NPT, 3 * NPT)])
        pltpu.sync_copy(ones_hbm, ones_v)
        pltpu.sync_copy(cols_hbm.at[pl.ds(wid * c_per_tile, c_per_tile)], idx_v)
        plsc.subcore_barrier()

        def body(j, carry):
            pltpu.sync_copy(ones_v, acc_sh.at[idx_v.at[j]], add=True)
            return carry

        lax.fori_loop(0, c_per_tile, body, 0)
        plsc.subcore_barrier()
        pltpu.sync_copy(acc_sh.at[pl.ds(sid * 3 * NPT, 3 * NPT)],
                        out_hbm.at[pl.ds(cid * 3 * NP + sid * 3 * NPT,
                                         3 * NPT)])

    return k


IBLK = 40
NSLOT = 4


@functools.lru_cache(maxsize=None)
def _make_scat_kernel(c_per_tile, dh):
    @functools.partial(
        pl.kernel,
        mesh=_mesh(),
        compiler_params=pltpu.CompilerParams(use_tc_tiling_on_sc=False),
        out_type=jax.ShapeDtypeStruct((NC, NP, dh), jnp.float32),
        scratch_types=[
            pltpu.VMEM((IBLK, CHUNK), jnp.int32),
            pltpu.VMEM((IBLK, CHUNK), jnp.int32),
            pltpu.VMEM((NSLOT, CHUNK, dh), jnp.float32),
            pltpu.VMEM_SHARED((NP, dh), jnp.float32),
            pltpu.VMEM_SHARED((NP, dh), jnp.float32),
        ] + [pltpu.SemaphoreType.DMA] * (2 * NSLOT),
    )
    def k(row_hbm, col_hbm, g_hbm, zeros_hbm, out_hbm,
          row_v, col_v, rows_v, acc_sh, tab_sh, *sems):
        cid = lax.axis_index("c")
        sid = lax.axis_index("s")
        base = sid * c_per_tile
        pltpu.sync_copy(g_hbm.at[cid, pl.ds(sid * NPT, NPT)],
                        tab_sh.at[pl.ds(sid * NPT, NPT)])
        gsrc = tab_sh
        gsrc_hbm = g_hbm.at[cid]
        pltpu.sync_copy(zeros_hbm, acc_sh.at[pl.ds(sid * NPT, NPT)])
        plsc.subcore_barrier()

        def make_grp(src):
            def grp_body(gi, carry2):
                j0 = gi * NSLOT
                descs = [
                    pltpu.async_copy(src.at[row_v.at[j0 + s]],
                                     rows_v.at[s], sems[s])
                    for s in range(NSLOT)
                ]
                sdescs = []
                for s in range(NSLOT):
                    descs[s].wait()
                    sdescs.append(pltpu.async_copy(
                        rows_v.at[s], acc_sh.at[col_v.at[j0 + s]],
                        sems[NSLOT + s], add=True))
                for s in range(NSLOT):
                    sdescs[s].wait()
                return carry2
            return grp_body

        def blk_body(b, carry):
            pltpu.sync_copy(row_hbm.at[pl.ds(base + b * IBLK, IBLK)], row_v)
            pltpu.sync_copy(col_hbm.at[pl.ds(base + b * IBLK, IBLK)], col_v)

            @pl.when(b % 3 == 0)
            def _():
                lax.fori_loop(0, IBLK // NSLOT, make_grp(gsrc_hbm), 0)

            @pl.when(b % 3 != 0)
            def _():
                lax.fori_loop(0, IBLK // NSLOT, make_grp(gsrc), 0)

            return carry

        lax.fori_loop(0, c_per_tile // IBLK, blk_body, 0)
        plsc.subcore_barrier()
        pltpu.sync_copy(acc_sh.at[pl.ds(sid * NPT, NPT)],
                        out_hbm.at[cid, pl.ds(sid * NPT, NPT)])

    return k


def _dinv(deg_ref, which):
    d = deg_ref[...]
    seg = (d[which * NP:which * NP + N]
           + d[3 * NP + which * NP:3 * NP + which * NP + N]
           + 1.0)
    return lax.rsqrt(seg)[:, None]


def _split(m, out_ref):
    d = m.shape[1] // 2
    mb = m
    pad = jnp.zeros((NP - N, d), jnp.float32)
    out_ref[0] = jnp.concatenate([mb[:, :d], pad], axis=0)
    out_ref[1] = jnp.concatenate([mb[:, d:], pad], axis=0)


def _unsplit(ref, rows):
    return jnp.concatenate([ref[0, :rows, :], ref[1, :rows, :]],
                           axis=1)


def _t1_body(deg_ref, x_ref, g1_ref):
    _split(x_ref[...] * _dinv(deg_ref, 0), g1_ref)


def _t2_body(deg_ref, g1_ref, p1_ref, w1_ref, b1_ref, w2_ref, g2_ref):
    dinv1 = _dinv(deg_ref, 0)
    agg1 = dinv1 * (_unsplit(p1_ref, N) + _unsplit(g1_ref, N))
    h1 = jax.nn.relu(jnp.dot(agg1, w1_ref[...],
                             preferred_element_type=jnp.float32)
                     + b1_ref[...][None, :])
    m2 = jnp.dot(h1, w2_ref[...], preferred_element_type=jnp.float32)
    _split(m2 * _dinv(deg_ref, 1), g2_ref)


def _t3_body(deg_ref, g2_ref, p2_ref, b2_ref, w3_ref, g3_ref):
    dinv2 = _dinv(deg_ref, 1)
    h2 = jax.nn.relu(dinv2 * (_unsplit(p2_ref, N) + _unsplit(g2_ref, N))
                     + b2_ref[...][None, :])
    m3 = jnp.dot(h2, w3_ref[...], preferred_element_type=jnp.float32)
    _split(m3 * _dinv(deg_ref, 2), g3_ref)


def _t4_body(deg_ref, g3_ref, p3_ref, b3_ref, hw1_ref, hb1_ref,
             bng_ref, bnb_ref, bnm_ref, bnv_ref, hw2_ref, hb2_ref,
             bw1_ref, bb1_ref, bw2_ref, bb2_ref,
             rw1_ref, rb1_ref, rw2_ref, rb2_ref,
             health_ref, bio_ref, risk_ref, h_ref):
    dinv3 = _dinv(deg_ref, 2)
    h = (dinv3 * (_unsplit(p3_ref, N) + _unsplit(g3_ref, N))
         + b3_ref[...][None, :])
    h_ref[...] = h

    hh = jax.nn.relu(jnp.dot(h, hw1_ref[...],
                             preferred_element_type=jnp.float32)
                     + hb1_ref[...][None, :])
    hh = ((hh - bnm_ref[...][None, :])
          * lax.rsqrt(bnv_ref[...][None, :] + 1e-5)
          * bng_ref[...][None, :] + bnb_ref[...][None, :])
    health = jnp.dot(hh, hw2_ref[...],
                     preferred_element_type=jnp.float32) + hb2_ref[...][None, :]
    health_ref[...] = jax.nn.softmax(health, axis=1)

    bio = jax.nn.relu(jnp.dot(h, bw1_ref[...],
                              preferred_element_type=jnp.float32)
                      + bb1_ref[...][None, :])
    bio = jnp.dot(bio, bw2_ref[...],
                  preferred_element_type=jnp.float32) + bb2_ref[...][None, :]
    bio_ref[...] = jax.nn.sigmoid(bio)

    h_next = jnp.concatenate([h[1:, :], h[:1, :]], axis=0)
    ht = jnp.concatenate([h, h_next], axis=1)
    risk = jax.nn.relu(jnp.dot(ht, rw1_ref[...],
                               preferred_element_type=jnp.float32)
                       + rb1_ref[...][None, :])
    risk = jnp.dot(risk, rw2_ref[...],
                   preferred_element_type=jnp.float32) + rb2_ref[...][None, :]
    risk_ref[...] = jax.nn.softmax(risk, axis=1)


def _tc_call(body, out_shapes, *args):
    return pl.pallas_call(
        body,
        out_shape=out_shapes,
    )(*args)


_C1 = 160
_C2 = 40
_CD = _C1 + 2 * _C2


@jax.jit
def kernel(x, edge_index_tree, edge_index_patch, edge_index_forest,
           W1, b1, W2, b2, W3, b3, hW1, hb1, bn_g, bn_b, bn_m, bn_v,
           hW2, hb2, bW1, bb1, bW2, bb2, rW1, rb1, rW2, rb2):
    r1, c1 = _pad_edges(edge_index_tree, NW * _C1)
    r2, c2 = _pad_edges(edge_index_patch, NW * _C2)
    r3, c3 = _pad_edges(edge_index_forest, NW * _C2)

    cd = jnp.concatenate([
        c1.reshape(-1), c2.reshape(-1) + NP, c3.reshape(-1) + 2 * NP,
        jnp.full((NW * _CD * CHUNK - NW * (_C1 + 2 * _C2) * CHUNK,), N,
                 jnp.int32),
    ]).reshape(NW * _CD, CHUNK)

    ones = jnp.ones((CHUNK,), jnp.float32)
    zeros_deg = jnp.zeros((3 * NPT,), jnp.float32)
    zeros64 = jnp.zeros((NPT, 64), jnp.float32)
    zeros16 = jnp.zeros((NPT, 16), jnp.float32)

    deg_part = _make_deg_kernel(_CD)(cd, ones, zeros_deg)

    g1 = _tc_call(_t1_body, jax.ShapeDtypeStruct((NC, NP, 64), jnp.float32),
                  deg_part, x)
    p1 = _make_scat_kernel(2 * _C1, 64)(r1, c1, g1, zeros64)
    g2 = _tc_call(_t2_body, jax.ShapeDtypeStruct((NC, NP, 64), jnp.float32),
                  deg_part, g1, p1, W1, b1, W2)

    p2 = _make_scat_kernel(2 * _C2, 64)(r2, c2, g2, zeros64)
    g3 = _tc_call(_t3_body, jax.ShapeDtypeStruct((NC, NP, 16), jnp.float32),
                  deg_part, g2, p2, b2, W3)

    p3 = _make_scat_kernel(2 * _C2, 16)(r3, c3, g3, zeros16)
    health, bio, risk, h = _tc_call(
        _t4_body,
        (jax.ShapeDtypeStruct((N, 5), jnp.float32),
         jax.ShapeDtypeStruct((N, 1), jnp.float32),
         jax.ShapeDtypeStruct((N, 3), jnp.float32),
         jax.ShapeDtypeStruct((N, 32), jnp.float32)),
        deg_part, g3, p3, b3, hW1, hb1, bn_g, bn_b, bn_m, bn_v, hW2, hb2,
        bW1, bb1, bW2, bb2, rW1, rb1, rW2, rb2)

    return health, bio, risk[:N - 1], h


# speedup vs baseline: 1.2669x; 1.2669x over previous
import functools
import jax
import jax.numpy as jnp
from jax import lax
from jax.experimental import pallas as pl
from jax.experimental.pallas import tpu as pltpu
from jax.experimental.pallas import tpu_sc as plsc

N = 10000
NP = 10240
NPT = NP // 16
NC = 2
NS = 16
NW = NC * NS
CHUNK = 128

@functools.lru_cache(maxsize=None)
def _mesh():
    return plsc.VectorSubcoreMesh(core_axis_name="c", subcore_axis_name="s")


def _pad_edges(ei, n_chunks_total):
    e = ei.shape[1]
    pad = n_chunks_total * CHUNK - e
    row = jnp.concatenate([ei[0], jnp.zeros((pad,), jnp.int32)])
    col = jnp.concatenate([ei[1], jnp.full((pad,), N, jnp.int32)])
    return (row.reshape(n_chunks_total, CHUNK),
            col.reshape(n_chunks_total, CHUNK))


@functools.lru_cache(maxsize=None)
def _make_deg_kernel(c_per_tile):
    @functools.partial(
        pl.kernel,
        mesh=_mesh(),
        out_type=jax.ShapeDtypeStruct((NC * 3 * NP,), jnp.float32),
        scratch_types=[
            pltpu.VMEM((c_per_tile, CHUNK), jnp.int32),
            pltpu.VMEM((CHUNK,), jnp.float32),
            pltpu.VMEM_SHARED((3 * NP,), jnp.float32),
        ],
    )
    def k(cols_hbm, ones_hbm, zeros_hbm, out_hbm, idx_v, ones_v, acc_sh):
        cid = lax.axis_index("c")
        sid = lax.axis_index("s")
        wid = sid * NC + cid
        pltpu.sync_copy(zeros_hbm, acc_sh.at[pl.ds(sid * 3 * NPT, 3 * NPT)])
        pltpu.sync_copy(ones_hbm, ones_v)
        pltpu.sync_copy(cols_hbm.at[pl.ds(wid * c_per_tile, c_per_tile)], idx_v)
        plsc.subcore_barrier()

        def body(j, carry):
            pltpu.sync_copy(ones_v, acc_sh.at[idx_v.at[j]], add=True)
            return carry

        lax.fori_loop(0, c_per_tile, body, 0)
        plsc.subcore_barrier()
        pltpu.sync_copy(acc_sh.at[pl.ds(sid * 3 * NPT, 3 * NPT)],
                        out_hbm.at[pl.ds(cid * 3 * NP + sid * 3 * NPT,
                                         3 * NPT)])

    return k


IBLK = 40
NSLOT = 4


@functools.lru_cache(maxsize=None)
def _make_scat_kernel(c_per_tile, dh):
    @functools.partial(
        pl.kernel,
        mesh=_mesh(),
        compiler_params=pltpu.CompilerParams(use_tc_tiling_on_sc=False),
        out_type=jax.ShapeDtypeStruct((NC, NP, dh), jnp.float32),
        scratch_types=[
            pltpu.VMEM((IBLK, CHUNK), jnp.int32),
            pltpu.VMEM((IBLK, CHUNK), jnp.int32),
            pltpu.VMEM((NSLOT, CHUNK, dh), jnp.float32),
            pltpu.VMEM_SHARED((NP, dh), jnp.float32),
            pltpu.VMEM_SHARED((NP, dh), jnp.float32),
        ] + [pltpu.SemaphoreType.DMA] * (2 * NSLOT),
    )
    def k(row_hbm, col_hbm, g_hbm, zeros_hbm, out_hbm,
          row_v, col_v, rows_v, acc_sh, tab_sh, *sems):
        cid = lax.axis_index("c")
        sid = lax.axis_index("s")
        base = sid * c_per_tile
        pltpu.sync_copy(g_hbm.at[cid, pl.ds(sid * NPT, NPT)],
                        tab_sh.at[pl.ds(sid * NPT, NPT)])
        gsrc = tab_sh
        pltpu.sync_copy(zeros_hbm, acc_sh.at[pl.ds(sid * NPT, NPT)])
        plsc.subcore_barrier()

        def make_grp(src):
            def grp_body(gi, carry2):
                j0 = gi * NSLOT
                descs = [
                    pltpu.async_copy(src.at[row_v.at[j0 + s]],
                                     rows_v.at[s], sems[s])
                    for s in range(NSLOT)
                ]
                sdescs = []
                for s in range(NSLOT):
                    descs[s].wait()
                    sdescs.append(pltpu.async_copy(
                        rows_v.at[s], acc_sh.at[col_v.at[j0 + s]],
                        sems[NSLOT + s], add=True))
                for s in range(NSLOT):
                    sdescs[s].wait()
                return carry2
            return grp_body

        def blk_body(b, carry):
            pltpu.sync_copy(row_hbm.at[pl.ds(base + b * IBLK, IBLK)], row_v)
            pltpu.sync_copy(col_hbm.at[pl.ds(base + b * IBLK, IBLK)], col_v)

            lax.fori_loop(0, IBLK // NSLOT, make_grp(gsrc), 0)
            return carry

        lax.fori_loop(0, c_per_tile // IBLK, blk_body, 0)
        plsc.subcore_barrier()
        pltpu.sync_copy(acc_sh.at[pl.ds(sid * NPT, NPT)],
                        out_hbm.at[cid, pl.ds(sid * NPT, NPT)])

    return k


def _dinv(deg_ref, which):
    d = deg_ref[...]
    seg = (d[which * NP:which * NP + N]
           + d[3 * NP + which * NP:3 * NP + which * NP + N]
           + 1.0)
    return lax.rsqrt(seg)[:, None]


def _split(m, out_ref):
    d = m.shape[1] // 2
    mb = m
    pad = jnp.zeros((NP - N, d), jnp.float32)
    out_ref[0] = jnp.concatenate([mb[:, :d], pad], axis=0)
    out_ref[1] = jnp.concatenate([mb[:, d:], pad], axis=0)


def _unsplit(ref, rows):
    return jnp.concatenate([ref[0, :rows, :], ref[1, :rows, :]],
                           axis=1)


def _t1_body(deg_ref, x_ref, g1_ref):
    _split(x_ref[...] * _dinv(deg_ref, 0), g1_ref)


def _t2_body(deg_ref, g1_ref, p1_ref, w1_ref, b1_ref, w2_ref, g2_ref):
    dinv1 = _dinv(deg_ref, 0)
    agg1 = dinv1 * (_unsplit(p1_ref, N) + _unsplit(g1_ref, N))
    h1 = jax.nn.relu(jnp.dot(agg1, w1_ref[...],
                             preferred_element_type=jnp.float32)
                     + b1_ref[...][None, :])
    m2 = jnp.dot(h1, w2_ref[...], preferred_element_type=jnp.float32)
    _split(m2 * _dinv(deg_ref, 1), g2_ref)


def _t3_body(deg_ref, g2_ref, p2_ref, b2_ref, w3_ref, g3_ref):
    dinv2 = _dinv(deg_ref, 1)
    h2 = jax.nn.relu(dinv2 * (_unsplit(p2_ref, N) + _unsplit(g2_ref, N))
                     + b2_ref[...][None, :])
    m3 = jnp.dot(h2, w3_ref[...], preferred_element_type=jnp.float32)
    _split(m3 * _dinv(deg_ref, 2), g3_ref)


def _t4_body(deg_ref, g3_ref, p3_ref, b3_ref, hw1_ref, hb1_ref,
             bng_ref, bnb_ref, bnm_ref, bnv_ref, hw2_ref, hb2_ref,
             bw1_ref, bb1_ref, bw2_ref, bb2_ref,
             rw1_ref, rb1_ref, rw2_ref, rb2_ref,
             health_ref, bio_ref, risk_ref, h_ref):
    dinv3 = _dinv(deg_ref, 2)
    h = (dinv3 * (_unsplit(p3_ref, N) + _unsplit(g3_ref, N))
         + b3_ref[...][None, :])
    h_ref[...] = h

    hh = jax.nn.relu(jnp.dot(h, hw1_ref[...],
                             preferred_element_type=jnp.float32)
                     + hb1_ref[...][None, :])
    hh = ((hh - bnm_ref[...][None, :])
          * lax.rsqrt(bnv_ref[...][None, :] + 1e-5)
          * bng_ref[...][None, :] + bnb_ref[...][None, :])
    health = jnp.dot(hh, hw2_ref[...],
                     preferred_element_type=jnp.float32) + hb2_ref[...][None, :]
    health_ref[...] = jax.nn.softmax(health, axis=1)

    bio = jax.nn.relu(jnp.dot(h, bw1_ref[...],
                              preferred_element_type=jnp.float32)
                      + bb1_ref[...][None, :])
    bio = jnp.dot(bio, bw2_ref[...],
                  preferred_element_type=jnp.float32) + bb2_ref[...][None, :]
    bio_ref[...] = jax.nn.sigmoid(bio)

    h_next = jnp.concatenate([h[1:, :], h[:1, :]], axis=0)
    ht = jnp.concatenate([h, h_next], axis=1)
    risk = jax.nn.relu(jnp.dot(ht, rw1_ref[...],
                               preferred_element_type=jnp.float32)
                       + rb1_ref[...][None, :])
    risk = jnp.dot(risk, rw2_ref[...],
                   preferred_element_type=jnp.float32) + rb2_ref[...][None, :]
    risk_ref[...] = jax.nn.softmax(risk, axis=1)


def _tc_call(body, out_shapes, *args):
    return pl.pallas_call(
        body,
        out_shape=out_shapes,
    )(*args)


_C1 = 160
_C2 = 40
_CD = _C1 + 2 * _C2


@jax.jit
def kernel(x, edge_index_tree, edge_index_patch, edge_index_forest,
           W1, b1, W2, b2, W3, b3, hW1, hb1, bn_g, bn_b, bn_m, bn_v,
           hW2, hb2, bW1, bb1, bW2, bb2, rW1, rb1, rW2, rb2):
    r1, c1 = _pad_edges(edge_index_tree, NW * _C1)
    r2, c2 = _pad_edges(edge_index_patch, NW * _C2)
    r3, c3 = _pad_edges(edge_index_forest, NW * _C2)

    cd = jnp.concatenate([
        c1.reshape(-1), c2.reshape(-1) + NP, c3.reshape(-1) + 2 * NP,
        jnp.full((NW * _CD * CHUNK - NW * (_C1 + 2 * _C2) * CHUNK,), N,
                 jnp.int32),
    ]).reshape(NW * _CD, CHUNK)

    ones = jnp.ones((CHUNK,), jnp.float32)
    zeros_deg = jnp.zeros((3 * NPT,), jnp.float32)
    zeros64 = jnp.zeros((NPT, 64), jnp.float32)
    zeros16 = jnp.zeros((NPT, 16), jnp.float32)

    deg_part = _make_deg_kernel(_CD)(cd, ones, zeros_deg)

    g1 = _tc_call(_t1_body, jax.ShapeDtypeStruct((NC, NP, 64), jnp.float32),
                  deg_part, x)
    p1 = _make_scat_kernel(2 * _C1, 64)(r1, c1, g1, zeros64)
    g2 = _tc_call(_t2_body, jax.ShapeDtypeStruct((NC, NP, 64), jnp.float32),
                  deg_part, g1, p1, W1, b1, W2)

    p2 = _make_scat_kernel(2 * _C2, 64)(r2, c2, g2, zeros64)
    g3 = _tc_call(_t3_body, jax.ShapeDtypeStruct((NC, NP, 16), jnp.float32),
                  deg_part, g2, p2, b2, W3)

    p3 = _make_scat_kernel(2 * _C2, 16)(r3, c3, g3, zeros16)
    health, bio, risk, h = _tc_call(
        _t4_body,
        (jax.ShapeDtypeStruct((N, 5), jnp.float32),
         jax.ShapeDtypeStruct((N, 1), jnp.float32),
         jax.ShapeDtypeStruct((N, 3), jnp.float32),
         jax.ShapeDtypeStruct((N, 32), jnp.float32)),
        deg_part, g3, p3, b3, hW1, hb1, bn_g, bn_b, bn_m, bn_v, hW2, hb2,
        bW1, bb1, bW2, bb2, rW1, rb1, rW2, rb2)

    return health, bio, risk[:N - 1], h


# speedup vs baseline: 1.2727x; 1.0046x over previous
import functools
import jax
import jax.numpy as jnp
from jax import lax
from jax.experimental import pallas as pl
from jax.experimental.pallas import tpu as pltpu
from jax.experimental.pallas import tpu_sc as plsc

N = 10000
NP = 10240
NPT = NP // 16
NC = 2
NS = 16
NW = NC * NS
CHUNK = 128

@functools.lru_cache(maxsize=None)
def _mesh():
    return plsc.VectorSubcoreMesh(core_axis_name="c", subcore_axis_name="s")


def _pad_edges(ei, n_chunks_total):
    e = ei.shape[1]
    pad = n_chunks_total * CHUNK - e
    row = jnp.concatenate([ei[0], jnp.zeros((pad,), jnp.int32)])
    col = jnp.concatenate([ei[1], jnp.full((pad,), N, jnp.int32)])
    return (row.reshape(n_chunks_total, CHUNK),
            col.reshape(n_chunks_total, CHUNK))


@functools.lru_cache(maxsize=None)
def _make_deg_kernel(c_per_tile):
    @functools.partial(
        pl.kernel,
        mesh=_mesh(),
        compiler_params=pltpu.CompilerParams(needs_layout_passes=False),
        out_type=jax.ShapeDtypeStruct((NW, 3 * NP // 128, 128), jnp.float32),
        scratch_types=[
            pltpu.VMEM((c_per_tile, CHUNK), jnp.int32),
            pltpu.VMEM((3 * NP // 128, 128), jnp.float32),
        ],
    )
    def k(cols_hbm, zeros_hbm, out_hbm, idx_v, deg_v):
        cid = lax.axis_index("c")
        sid = lax.axis_index("s")
        wid = sid * NC + cid
        pltpu.sync_copy(zeros_hbm, deg_v)
        pltpu.sync_copy(cols_hbm.at[pl.ds(wid * c_per_tile, c_per_tile)],
                        idx_v)
        ones = jnp.full((16,), 1.0, jnp.float32)

        def chunk(j, carry):
            for l in range(CHUNK // 16):
                idx16 = idx_v[j, pl.ds(l * 16, 16)]
                r = lax.shift_right_logical(idx16, 7)
                c = lax.bitwise_and(idx16, 127)
                plsc.addupdate_scatter(deg_v, [r, c], ones)
            return carry

        lax.fori_loop(0, c_per_tile, chunk, 0)
        pltpu.sync_copy(deg_v, out_hbm.at[wid])

    return k


IBLK = 40
NSLOT = 4


@functools.lru_cache(maxsize=None)
def _make_scat_kernel(c_per_tile, dh):
    @functools.partial(
        pl.kernel,
        mesh=_mesh(),
        compiler_params=pltpu.CompilerParams(use_tc_tiling_on_sc=False),
        out_type=jax.ShapeDtypeStruct((NC, NP, dh), jnp.float32),
        scratch_types=[
            pltpu.VMEM((IBLK, CHUNK), jnp.int32),
            pltpu.VMEM((IBLK, CHUNK), jnp.int32),
            pltpu.VMEM((NSLOT, CHUNK, dh), jnp.float32),
            pltpu.VMEM_SHARED((NP, dh), jnp.float32),
            pltpu.VMEM_SHARED((NP, dh), jnp.float32),
        ] + [pltpu.SemaphoreType.DMA] * (2 * NSLOT),
    )
    def k(row_hbm, col_hbm, g_hbm, zeros_hbm, out_hbm,
          row_v, col_v, rows_v, acc_sh, tab_sh, *sems):
        cid = lax.axis_index("c")
        sid = lax.axis_index("s")
        base = sid * c_per_tile
        pltpu.sync_copy(g_hbm.at[cid, pl.ds(sid * NPT, NPT)],
                        tab_sh.at[pl.ds(sid * NPT, NPT)])
        gsrc = tab_sh
        pltpu.sync_copy(zeros_hbm, acc_sh.at[pl.ds(sid * NPT, NPT)])
        plsc.subcore_barrier()

        def make_grp(src):
            def grp_body(gi, carry2):
                j0 = gi * NSLOT
                descs = [
                    pltpu.async_copy(src.at[row_v.at[j0 + s]],
                                     rows_v.at[s], sems[s])
                    for s in range(NSLOT)
                ]
                sdescs = []
                for s in range(NSLOT):
                    descs[s].wait()
                    sdescs.append(pltpu.async_copy(
                        rows_v.at[s], acc_sh.at[col_v.at[j0 + s]],
                        sems[NSLOT + s], add=True))
                for s in range(NSLOT):
                    sdescs[s].wait()
                return carry2
            return grp_body

        def blk_body(b, carry):
            pltpu.sync_copy(row_hbm.at[pl.ds(base + b * IBLK, IBLK)], row_v)
            pltpu.sync_copy(col_hbm.at[pl.ds(base + b * IBLK, IBLK)], col_v)

            lax.fori_loop(0, IBLK // NSLOT, make_grp(gsrc), 0)
            return carry

        lax.fori_loop(0, c_per_tile // IBLK, blk_body, 0)
        plsc.subcore_barrier()
        pltpu.sync_copy(acc_sh.at[pl.ds(sid * NPT, NPT)],
                        out_hbm.at[cid, pl.ds(sid * NPT, NPT)])

    return k


_DR = NP // 128


def _dinv_from(ds, which):
    seg = ds[which * _DR:(which + 1) * _DR, :].reshape(NP)[:N] + 1.0
    return lax.rsqrt(seg)[:, None]


def _dinv(ds_ref, which):
    return _dinv_from(ds_ref[...], which)


def _split(m, out_ref):
    d = m.shape[1] // 2
    mb = m
    pad = jnp.zeros((NP - N, d), jnp.float32)
    out_ref[0] = jnp.concatenate([mb[:, :d], pad], axis=0)
    out_ref[1] = jnp.concatenate([mb[:, d:], pad], axis=0)


def _unsplit(ref, rows):
    return jnp.concatenate([ref[0, :rows, :], ref[1, :rows, :]],
                           axis=1)


def _t1_body(degp_ref, x_ref, g1_ref, ds_ref):
    ds = jnp.sum(degp_ref[...], axis=0)
    ds_ref[...] = ds
    _split(x_ref[...] * _dinv_from(ds, 0), g1_ref)


def _t2_body(deg_ref, g1_ref, p1_ref, w1_ref, b1_ref, w2_ref, g2_ref):
    dinv1 = _dinv(deg_ref, 0)
    agg1 = dinv1 * (_unsplit(p1_ref, N) + _unsplit(g1_ref, N))
    h1 = jax.nn.relu(jnp.dot(agg1, w1_ref[...],
                             preferred_element_type=jnp.float32)
                     + b1_ref[...][None, :])
    m2 = jnp.dot(h1, w2_ref[...], preferred_element_type=jnp.float32)
    _split(m2 * _dinv(deg_ref, 1), g2_ref)


def _t3_body(deg_ref, g2_ref, p2_ref, b2_ref, w3_ref, g3_ref):
    dinv2 = _dinv(deg_ref, 1)
    h2 = jax.nn.relu(dinv2 * (_unsplit(p2_ref, N) + _unsplit(g2_ref, N))
                     + b2_ref[...][None, :])
    m3 = jnp.dot(h2, w3_ref[...], preferred_element_type=jnp.float32)
    _split(m3 * _dinv(deg_ref, 2), g3_ref)


def _t4_body(deg_ref, g3_ref, p3_ref, b3_ref, hw1_ref, hb1_ref,
             bng_ref, bnb_ref, bnm_ref, bnv_ref, hw2_ref, hb2_ref,
             bw1_ref, bb1_ref, bw2_ref, bb2_ref,
             rw1_ref, rb1_ref, rw2_ref, rb2_ref,
             health_ref, bio_ref, risk_ref, h_ref):
    dinv3 = _dinv(deg_ref, 2)
    h = (dinv3 * (_unsplit(p3_ref, N) + _unsplit(g3_ref, N))
         + b3_ref[...][None, :])
    h_ref[...] = h

    hh = jax.nn.relu(jnp.dot(h, hw1_ref[...],
                             preferred_element_type=jnp.float32)
                     + hb1_ref[...][None, :])
    hh = ((hh - bnm_ref[...][None, :])
          * lax.rsqrt(bnv_ref[...][None, :] + 1e-5)
          * bng_ref[...][None, :] + bnb_ref[...][None, :])
    health = jnp.dot(hh, hw2_ref[...],
                     preferred_element_type=jnp.float32) + hb2_ref[...][None, :]
    health_ref[...] = jax.nn.softmax(health, axis=1)

    bio = jax.nn.relu(jnp.dot(h, bw1_ref[...],
                              preferred_element_type=jnp.float32)
                      + bb1_ref[...][None, :])
    bio = jnp.dot(bio, bw2_ref[...],
                  preferred_element_type=jnp.float32) + bb2_ref[...][None, :]
    bio_ref[...] = jax.nn.sigmoid(bio)

    h_next = jnp.concatenate([h[1:, :], h[:1, :]], axis=0)
    ht = jnp.concatenate([h, h_next], axis=1)
    risk = jax.nn.relu(jnp.dot(ht, rw1_ref[...],
                               preferred_element_type=jnp.float32)
                       + rb1_ref[...][None, :])
    risk = jnp.dot(risk, rw2_ref[...],
                   preferred_element_type=jnp.float32) + rb2_ref[...][None, :]
    risk_ref[...] = jax.nn.softmax(risk, axis=1)


def _tc_call(body, out_shapes, *args):
    return pl.pallas_call(
        body,
        out_shape=out_shapes,
    )(*args)


_C1 = 160
_C2 = 40
_CD = _C1 + 2 * _C2


@jax.jit
def kernel(x, edge_index_tree, edge_index_patch, edge_index_forest,
           W1, b1, W2, b2, W3, b3, hW1, hb1, bn_g, bn_b, bn_m, bn_v,
           hW2, hb2, bW1, bb1, bW2, bb2, rW1, rb1, rW2, rb2):
    r1, c1 = _pad_edges(edge_index_tree, NW * _C1)
    r2, c2 = _pad_edges(edge_index_patch, NW * _C2)
    r3, c3 = _pad_edges(edge_index_forest, NW * _C2)

    cd = jnp.concatenate([
        c1.reshape(-1), c2.reshape(-1) + NP, c3.reshape(-1) + 2 * NP,
        jnp.full((NW * _CD * CHUNK - NW * (_C1 + 2 * _C2) * CHUNK,), N,
                 jnp.int32),
    ]).reshape(NW * _CD, CHUNK)

    zeros_deg = jnp.zeros((3 * NP // 128, 128), jnp.float32)
    zeros64 = jnp.zeros((NPT, 64), jnp.float32)
    zeros16 = jnp.zeros((NPT, 16), jnp.float32)

    deg_part = _make_deg_kernel(_CD)(cd, zeros_deg)

    g1, degsum = _tc_call(
        _t1_body,
        (jax.ShapeDtypeStruct((NC, NP, 64), jnp.float32),
         jax.ShapeDtypeStruct((3 * NP // 128, 128), jnp.float32)),
        deg_part, x)
    p1 = _make_scat_kernel(2 * _C1, 64)(r1, c1, g1, zeros64)
    g2 = _tc_call(_t2_body, jax.ShapeDtypeStruct((NC, NP, 64), jnp.float32),
                  degsum, g1, p1, W1, b1, W2)

    p2 = _make_scat_kernel(2 * _C2, 64)(r2, c2, g2, zeros64)
    g3 = _tc_call(_t3_body, jax.ShapeDtypeStruct((NC, NP, 16), jnp.float32),
                  degsum, g2, p2, b2, W3)

    p3 = _make_scat_kernel(2 * _C2, 16)(r3, c3, g3, zeros16)
    health, bio, risk, h = _tc_call(
        _t4_body,
        (jax.ShapeDtypeStruct((N, 5), jnp.float32),
         jax.ShapeDtypeStruct((N, 1), jnp.float32),
         jax.ShapeDtypeStruct((N, 3), jnp.float32),
         jax.ShapeDtypeStruct((N, 32), jnp.float32)),
        degsum, g3, p3, b3, hW1, hb1, bn_g, bn_b, bn_m, bn_v, hW2, hb2,
        bW1, bb1, bW2, bb2, rW1, rb1, rW2, rb2)

    return health, bio, risk[:N - 1], h
